# Initial kernel scaffold; baseline (speedup 1.0000x reference)
import jax
import jax.numpy as jnp
from jax.experimental import pallas as pl
from jax.experimental.pallas import tpu as pltpu

_B, _N, _C, _H, _W = 8, 64, 8, 64, 64
_M = 32
_HW = _H * _W
_NC = _C - 1


def _ot_kernel(gt_ref, s_ref, prob_ref, pg_ref, out_ref, ratio_ref, acc_ref):
    b = pl.program_id(0)
    ci = pl.program_id(1)

    s = s_ref[0, :, 0, :]
    g = gt_ref[0]

    mask = (g == (ci + 1)).astype(jnp.float32)

    inter = jax.lax.dot_general(
        s, mask,
        dimension_numbers=(((1,), (1,)), ((), ())),
        preferred_element_type=jnp.float32,
    )
    s_sum = jnp.sum(s, axis=1, keepdims=True)
    g_sum = jnp.sum(mask, axis=1, keepdims=True)
    union = s_sum + g_sum.reshape(1, _M) - inter
    term = (inter + 1.0) / (union + 1.0)

    @pl.when(ci == 0)
    def _():
        ratio_ref[...] = term

    @pl.when(ci != 0)
    def _():
        ratio_ref[...] += term

    @pl.when(ci == _NC - 1)
    def _():
        cost = 1.0 - ratio_ref[...] * (1.0 / _NC)
        minv = jnp.min(cost, axis=0, keepdims=True)
        iota_n = jax.lax.broadcasted_iota(jnp.int32, (_N, _M), 0)
        cand = jnp.where(cost <= minv, iota_n, _N)
        idx = jnp.min(cand, axis=0, keepdims=True)
        onehot = (iota_n == idx).astype(jnp.float32)

        pg = pg_ref[0]
        seg_b = jnp.sum(minv * pg)

        target = jnp.sum(onehot * pg, axis=1, keepdims=True)
        p = prob_ref[0]
        safe_t = jnp.where(target > 0, target, 1.0)
        kl_elem = jnp.where(
            target > 0, target * (jnp.log(safe_t) - jnp.log(p + 1e-8)), 0.0
        )
        kl_b = jnp.sum(kl_elem)

        @pl.when(b == 0)
        def _():
            acc_ref[0] = seg_b
            acc_ref[1] = kl_b

        @pl.when(b != 0)
        def _():
            acc_ref[0] += seg_b
            acc_ref[1] += kl_b

        @pl.when(b == _B - 1)
        def _():
            seg_loss = acc_ref[0] * (1.0 / _B)
            kl_loss = acc_ref[1] * (1.0 / (_B * _N))
            out_ref[0] = seg_loss + kl_loss
            out_ref[1] = seg_loss
            out_ref[2] = kl_loss


def kernel(gt_arr, sample_arr, prob, prob_gt, sample_shape):
    del sample_shape
    gt = gt_arr.reshape(_B, _M, _HW)
    s = sample_arr.reshape(_B, _N, _C, _HW)
    p = prob.reshape(_B, _N, 1)
    pg = prob_gt.reshape(_B, 1, _M)

    out = pl.pallas_call(
        _ot_kernel,
        grid=(_B, _NC),
        in_specs=[
            pl.BlockSpec((1, _M, _HW), lambda b, c: (b, 0, 0)),
            pl.BlockSpec((1, _N, 1, _HW), lambda b, c: (b, 0, c + 1, 0)),
            pl.BlockSpec((1, _N, 1), lambda b, c: (b, 0, 0)),
            pl.BlockSpec((1, 1, _M), lambda b, c: (b, 0, 0)),
        ],
        out_specs=pl.BlockSpec(memory_space=pltpu.SMEM),
        out_shape=jax.ShapeDtypeStruct((3,), jnp.float32),
        scratch_shapes=[
            pltpu.VMEM((_N, _M), jnp.float32),
            pltpu.SMEM((2,), jnp.float32),
        ],
    )(gt, s, p, pg)
    return (out[0], out[1], out[2])


# speedup vs baseline: 1.2558x; 1.2558x over previous
import jax
import jax.numpy as jnp
from jax.experimental import pallas as pl
from jax.experimental.pallas import tpu as pltpu

_B, _N, _C, _H, _W = 8, 64, 8, 64, 64
_M = 32
_HW = _H * _W
_NC = _C - 1


def _ot_kernel(gt_ref, s_ref, prob_ref, pg_ref, out_ref, ratio_ref, acc_ref):
    b = pl.program_id(0)
    ci = pl.program_id(1)

    s = s_ref[0]
    g = gt_ref[0]

    mask = (g == (ci + 1)).astype(jnp.float32)

    inter = jax.lax.dot_general(
        s, mask,
        dimension_numbers=(((1,), (1,)), ((), ())),
        preferred_element_type=jnp.float32,
    )
    s_sum = jnp.sum(s, axis=1, keepdims=True)
    g_sum = jnp.sum(mask, axis=1, keepdims=True)
    union = s_sum + g_sum.reshape(1, _M) - inter
    term = (inter + 1.0) / (union + 1.0)

    @pl.when(ci == 0)
    def _():
        ratio_ref[...] = term

    @pl.when(ci != 0)
    def _():
        ratio_ref[...] += term

    @pl.when(ci == _NC - 1)
    def _():
        cost = 1.0 - ratio_ref[...] * (1.0 / _NC)
        minv = jnp.min(cost, axis=0, keepdims=True)
        iota_n = jax.lax.broadcasted_iota(jnp.int32, (_N, _M), 0)
        cand = jnp.where(cost <= minv, iota_n, _N)
        idx = jnp.min(cand, axis=0, keepdims=True)
        onehot = (iota_n == idx).astype(jnp.float32)

        pg = pg_ref[0]
        seg_b = jnp.sum(minv * pg)

        target = jnp.sum(onehot * pg, axis=1, keepdims=True)
        p = prob_ref[0]
        safe_t = jnp.where(target > 0, target, 1.0)
        kl_elem = jnp.where(
            target > 0, target * (jnp.log(safe_t) - jnp.log(p + 1e-8)), 0.0
        )
        kl_b = jnp.sum(kl_elem)

        @pl.when(b == 0)
        def _():
            acc_ref[0] = seg_b
            acc_ref[1] = kl_b

        @pl.when(b != 0)
        def _():
            acc_ref[0] += seg_b
            acc_ref[1] += kl_b

        @pl.when(b == _B - 1)
        def _():
            seg_loss = acc_ref[0] * (1.0 / _B)
            kl_loss = acc_ref[1] * (1.0 / (_B * _N))
            out_ref[0] = seg_loss + kl_loss
            out_ref[1] = seg_loss
            out_ref[2] = kl_loss


def kernel(gt_arr, sample_arr, prob, prob_gt, sample_shape):
    del sample_shape
    gt = gt_arr.reshape(_B, _M, _HW)
    s = sample_arr.reshape(_B, _N, _C * _HW)
    p = prob.reshape(_B, _N, 1)
    pg = prob_gt.reshape(_B, 1, _M)

    out = pl.pallas_call(
        _ot_kernel,
        grid=(_B, _NC),
        in_specs=[
            pl.BlockSpec((1, _M, _HW), lambda b, c: (b, 0, 0)),
            pl.BlockSpec((1, _N, _HW), lambda b, c: (b, 0, c + 1)),
            pl.BlockSpec((1, _N, 1), lambda b, c: (b, 0, 0)),
            pl.BlockSpec((1, 1, _M), lambda b, c: (b, 0, 0)),
        ],
        out_specs=pl.BlockSpec(memory_space=pltpu.SMEM),
        out_shape=jax.ShapeDtypeStruct((3,), jnp.float32),
        scratch_shapes=[
            pltpu.VMEM((_N, _M), jnp.float32),
            pltpu.SMEM((2,), jnp.float32),
        ],
    )(gt, s, p, pg)
    return (out[0], out[1], out[2])


# speedup vs baseline: 2.4536x; 1.9539x over previous
import jax
import jax.numpy as jnp
from jax.experimental import pallas as pl
from jax.experimental.pallas import tpu as pltpu

_B, _N, _C, _H, _W = 8, 64, 8, 64, 64
_M = 32
_HW = _H * _W
_NC = _C - 1


def _ot_kernel(gt_ref, s_ref, prob_ref, pg_ref, out_ref, ratio_ref, g2_ref,
               acc_ref):
    b = pl.program_id(0)
    ci = pl.program_id(1)

    @pl.when(ci == 0)
    def _():
        g2_ref[...] = gt_ref[0].reshape(_M, _HW)

    s = s_ref[0, :, 0].reshape(_N, _HW)
    g = g2_ref[...]

    mask = (g == (ci + 1)).astype(jnp.float32)

    inter = jax.lax.dot_general(
        s, mask,
        dimension_numbers=(((1,), (1,)), ((), ())),
        preferred_element_type=jnp.float32,
    )
    s_sum = jnp.sum(s, axis=1, keepdims=True)
    g_sum = jnp.sum(mask, axis=1, keepdims=True)
    union = s_sum + g_sum.reshape(1, _M) - inter
    term = (inter + 1.0) / (union + 1.0)

    @pl.when(ci == 0)
    def _():
        ratio_ref[...] = term

    @pl.when(ci != 0)
    def _():
        ratio_ref[...] += term

    @pl.when(ci == _NC - 1)
    def _():
        cost = 1.0 - ratio_ref[...] * (1.0 / _NC)
        minv = jnp.min(cost, axis=0, keepdims=True)
        iota_n = jax.lax.broadcasted_iota(jnp.int32, (_N, _M), 0)
        cand = jnp.where(cost <= minv, iota_n, _N)
        idx = jnp.min(cand, axis=0, keepdims=True)
        onehot = (iota_n == idx).astype(jnp.float32)

        pg = pg_ref[0]
        seg_b = jnp.sum(minv * pg)

        target = jnp.sum(onehot * pg, axis=1, keepdims=True)
        p = prob_ref[0]
        safe_t = jnp.where(target > 0, target, 1.0)
        kl_elem = jnp.where(
            target > 0, target * (jnp.log(safe_t) - jnp.log(p + 1e-8)), 0.0
        )
        kl_b = jnp.sum(kl_elem)

        @pl.when(b == 0)
        def _():
            acc_ref[0] = seg_b
            acc_ref[1] = kl_b

        @pl.when(b != 0)
        def _():
            acc_ref[0] += seg_b
            acc_ref[1] += kl_b

        @pl.when(b == _B - 1)
        def _():
            seg_loss = acc_ref[0] * (1.0 / _B)
            kl_loss = acc_ref[1] * (1.0 / (_B * _N))
            out_ref[0] = seg_loss + kl_loss
            out_ref[1] = seg_loss
            out_ref[2] = kl_loss


def kernel(gt_arr, sample_arr, prob, prob_gt, sample_shape):
    del sample_shape
    p = prob.reshape(_B, _N, 1)
    pg = prob_gt.reshape(_B, 1, _M)

    out = pl.pallas_call(
        _ot_kernel,
        grid=(_B, _NC),
        in_specs=[
            pl.BlockSpec((1, _M, _H, _W), lambda b, c: (b, 0, 0, 0)),
            pl.BlockSpec((1, _N, 1, _H, _W), lambda b, c: (b, 0, c + 1, 0, 0)),
            pl.BlockSpec((1, _N, 1), lambda b, c: (b, 0, 0)),
            pl.BlockSpec((1, 1, _M), lambda b, c: (b, 0, 0)),
        ],
        out_specs=pl.BlockSpec(memory_space=pltpu.SMEM),
        out_shape=jax.ShapeDtypeStruct((3,), jnp.float32),
        scratch_shapes=[
            pltpu.VMEM((_N, _M), jnp.float32),
            pltpu.VMEM((_M, _HW), jnp.int32),
            pltpu.SMEM((2,), jnp.float32),
        ],
    )(gt_arr, sample_arr, p, pg)
    return (out[0], out[1], out[2])


# speedup vs baseline: 2.5605x; 1.0436x over previous
import jax
import jax.numpy as jnp
from jax.experimental import pallas as pl
from jax.experimental.pallas import tpu as pltpu

_B, _N, _C, _H, _W = 8, 64, 8, 64, 64
_M = 32
_HW = _H * _W
_NC = _C - 1
_NA = _N + 8
_MA = _M + 8


def _ot_kernel(gt_ref, s_ref, prob_ref, pg_ref, out_ref, ratio_ref, g2_ref,
               sa_ref, acc_ref):
    b = pl.program_id(0)
    ci = pl.program_id(1)

    @pl.when((b == 0) & (ci == 0))
    def _():
        r8 = jax.lax.broadcasted_iota(jnp.int32, (8, _HW), 0)
        sa_ref[_N:_NA, :] = jnp.where(r8 == 0, 1.0, 0.0).astype(jnp.bfloat16)
        g2_ref[_M + 1:_MA, :] = jnp.full((7, _HW), -1, jnp.int32)

    @pl.when(ci == 0)
    def _():
        g2_ref[0:_M, :] = gt_ref[0].reshape(_M, _HW)

    g2_ref[_M:_M + 1, :] = jnp.zeros((1, _HW), jnp.int32) + (ci + 1)

    sa_ref[0:_N, :] = s_ref[0, :, 0].astype(jnp.bfloat16).reshape(_N, _HW)
    mask = (g2_ref[...] == (ci + 1)).astype(jnp.bfloat16)

    out = jax.lax.dot_general(
        sa_ref[...], mask,
        dimension_numbers=(((1,), (1,)), ((), ())),
        preferred_element_type=jnp.float32,
    )
    inter = out[0:_N, 0:_M]
    s_sum = out[0:_N, _M:_M + 1]
    g_sum = out[_N:_N + 1, 0:_M]
    union = s_sum + g_sum - inter
    term = (inter + 1.0) / (union + 1.0)

    @pl.when(ci == 0)
    def _():
        ratio_ref[...] = term

    @pl.when(ci != 0)
    def _():
        ratio_ref[...] += term

    @pl.when(ci == _NC - 1)
    def _():
        cost = 1.0 - ratio_ref[...] * (1.0 / _NC)
        minv = jnp.min(cost, axis=0, keepdims=True)
        iota_n = jax.lax.broadcasted_iota(jnp.int32, (_N, _M), 0)
        cand = jnp.where(cost <= minv, iota_n, _N)
        idx = jnp.min(cand, axis=0, keepdims=True)
        onehot = (iota_n == idx).astype(jnp.float32)

        pg = pg_ref[0]
        seg_b = jnp.sum(minv * pg)

        target = jnp.sum(onehot * pg, axis=1, keepdims=True)
        p = prob_ref[0]
        safe_t = jnp.where(target > 0, target, 1.0)
        kl_elem = jnp.where(
            target > 0, target * (jnp.log(safe_t) - jnp.log(p + 1e-8)), 0.0
        )
        kl_b = jnp.sum(kl_elem)

        @pl.when(b == 0)
        def _():
            acc_ref[0] = seg_b
            acc_ref[1] = kl_b

        @pl.when(b != 0)
        def _():
            acc_ref[0] += seg_b
            acc_ref[1] += kl_b

        @pl.when(b == _B - 1)
        def _():
            seg_loss = acc_ref[0] * (1.0 / _B)
            kl_loss = acc_ref[1] * (1.0 / (_B * _N))
            out_ref[0] = seg_loss + kl_loss
            out_ref[1] = seg_loss
            out_ref[2] = kl_loss


def kernel(gt_arr, sample_arr, prob, prob_gt, sample_shape):
    del sample_shape
    p = prob.reshape(_B, _N, 1)
    pg = prob_gt.reshape(_B, 1, _M)

    out = pl.pallas_call(
        _ot_kernel,
        grid=(_B, _NC),
        in_specs=[
            pl.BlockSpec((1, _M, _H, _W), lambda b, c: (b, 0, 0, 0)),
            pl.BlockSpec((1, _N, 1, _H, _W), lambda b, c: (b, 0, c + 1, 0, 0)),
            pl.BlockSpec((1, _N, 1), lambda b, c: (b, 0, 0)),
            pl.BlockSpec((1, 1, _M), lambda b, c: (b, 0, 0)),
        ],
        out_specs=pl.BlockSpec(memory_space=pltpu.SMEM),
        out_shape=jax.ShapeDtypeStruct((3,), jnp.float32),
        scratch_shapes=[
            pltpu.VMEM((_N, _M), jnp.float32),
            pltpu.VMEM((_MA, _HW), jnp.int32),
            pltpu.VMEM((_NA, _HW), jnp.bfloat16),
            pltpu.SMEM((2,), jnp.float32),
        ],
    )(gt_arr, sample_arr, p, pg)
    return (out[0], out[1], out[2])


# speedup vs baseline: 3.5162x; 1.3733x over previous
import jax
import jax.numpy as jnp
from jax.experimental import pallas as pl
from jax.experimental.pallas import tpu as pltpu

_B, _N, _C, _H, _W = 8, 64, 8, 64, 64
_M = 32
_HW = _H * _W
_NC = _C - 1
_NA = _N + 8
_MA = _M + 8


def _ot_kernel(gt_ref, sa_in, sb_in, prob_ref, pg_ref, out_ref, ratio_ref,
               g2_ref, sa_ref, acc_ref):
    b = pl.program_id(0)
    j = pl.program_id(1)

    @pl.when((b == 0) & (j == 0))
    def _():
        r8 = jax.lax.broadcasted_iota(jnp.int32, (8, _HW), 0)
        sa_ref[_N:_NA, :] = jnp.where(r8 == 0, 1.0, 0.0).astype(jnp.bfloat16)
        g2_ref[_M + 1:_MA, :] = jnp.full((7, _HW), -1, jnp.int32)

    @pl.when(j == 0)
    def _():
        g2_ref[0:_M, :] = gt_ref[0].reshape(_M, _HW)

    def channel(k, s_in, ki):
        c = j * 4 + k
        g2_ref[_M:_M + 1, :] = jnp.zeros((1, _HW), jnp.int32) + c
        sa_ref[0:_N, :] = (
            s_in[0, :, ki].astype(jnp.bfloat16).reshape(_N, _HW)
        )
        mask = (g2_ref[...] == c).astype(jnp.bfloat16)
        out = jax.lax.dot_general(
            sa_ref[...], mask,
            dimension_numbers=(((1,), (1,)), ((), ())),
            preferred_element_type=jnp.float32,
        )
        inter = out[0:_N, 0:_M]
        s_sum = out[0:_N, _M:_M + 1]
        g_sum = out[_N:_N + 1, 0:_M]
        union = s_sum + g_sum - inter
        return (inter + 1.0) / (union + 1.0)

    @pl.when(j == 0)
    def _():
        ratio_ref[...] = channel(1, sa_in, 1)
        ratio_ref[...] += channel(2, sb_in, 0)
        ratio_ref[...] += channel(3, sb_in, 1)

    @pl.when(j == 1)
    def _():
        ratio_ref[...] += channel(0, sa_in, 0)
        ratio_ref[...] += channel(1, sa_in, 1)
        ratio_ref[...] += channel(2, sb_in, 0)
        ratio_ref[...] += channel(3, sb_in, 1)

        cost = 1.0 - ratio_ref[...] * (1.0 / _NC)
        minv = jnp.min(cost, axis=0, keepdims=True)
        iota_n = jax.lax.broadcasted_iota(jnp.int32, (_N, _M), 0)
        cand = jnp.where(cost <= minv, iota_n, _N)
        idx = jnp.min(cand, axis=0, keepdims=True)
        onehot = (iota_n == idx).astype(jnp.float32)

        pg = pg_ref[0]
        seg_b = jnp.sum(minv * pg)

        target = jnp.sum(onehot * pg, axis=1, keepdims=True)
        p = prob_ref[0]
        safe_t = jnp.where(target > 0, target, 1.0)
        kl_elem = jnp.where(
            target > 0, target * (jnp.log(safe_t) - jnp.log(p + 1e-8)), 0.0
        )
        kl_b = jnp.sum(kl_elem)

        @pl.when(b == 0)
        def _():
            acc_ref[0] = seg_b
            acc_ref[1] = kl_b

        @pl.when(b != 0)
        def _():
            acc_ref[0] += seg_b
            acc_ref[1] += kl_b

        @pl.when(b == _B - 1)
        def _():
            seg_loss = acc_ref[0] * (1.0 / _B)
            kl_loss = acc_ref[1] * (1.0 / (_B * _N))
            out_ref[0] = seg_loss + kl_loss
            out_ref[1] = seg_loss
            out_ref[2] = kl_loss


def kernel(gt_arr, sample_arr, prob, prob_gt, sample_shape):
    del sample_shape
    p = prob.reshape(_B, _N, 1)
    pg = prob_gt.reshape(_B, 1, _M)

    out = pl.pallas_call(
        _ot_kernel,
        grid=(_B, 2),
        in_specs=[
            pl.BlockSpec((1, _M, _H, _W), lambda b, j: (b, 0, 0, 0)),
            pl.BlockSpec((1, _N, 2, _H, _W), lambda b, j: (b, 0, 2 * j, 0, 0)),
            pl.BlockSpec((1, _N, 2, _H, _W),
                         lambda b, j: (b, 0, 2 * j + 1, 0, 0)),
            pl.BlockSpec((1, _N, 1), lambda b, j: (b, 0, 0)),
            pl.BlockSpec((1, 1, _M), lambda b, j: (b, 0, 0)),
        ],
        out_specs=pl.BlockSpec(memory_space=pltpu.SMEM),
        out_shape=jax.ShapeDtypeStruct((3,), jnp.float32),
        scratch_shapes=[
            pltpu.VMEM((_N, _M), jnp.float32),
            pltpu.VMEM((_MA, _HW), jnp.int32),
            pltpu.VMEM((_NA, _HW), jnp.bfloat16),
            pltpu.SMEM((2,), jnp.float32),
        ],
    )(gt_arr, sample_arr, sample_arr, p, pg)
    return (out[0], out[1], out[2])


# speedup vs baseline: 3.9335x; 1.1187x over previous
import jax
import jax.numpy as jnp
from jax.experimental import pallas as pl
from jax.experimental.pallas import tpu as pltpu

_B, _N, _C, _H, _W = 8, 64, 8, 64, 64
_M = 32
_HW = _H * _W
_NC = _C - 1
_NA = _N + 8
_MA = _M + 8


def _ot_kernel(gt_ref, s1_ref, s2_ref, s4_ref, prob_ref, pg_ref, out_ref,
               g2_ref, sa_ref, acc_ref):
    b = pl.program_id(0)

    @pl.when(b == 0)
    def _():
        r8 = jax.lax.broadcasted_iota(jnp.int32, (8, _HW), 0)
        sa_ref[_N:_NA, :] = jnp.where(r8 == 0, 1.0, 0.0).astype(jnp.bfloat16)
        g2_ref[_M + 1:_MA, :] = jnp.full((7, _HW), -1, jnp.int32)

    g2_ref[0:_M, :] = gt_ref[0].reshape(_M, _HW)

    def channel(c, s_in, ki):
        g2_ref[_M:_M + 1, :] = jnp.full((1, _HW), c, jnp.int32)
        sa_ref[0:_N, :] = (
            s_in[0, :, ki].astype(jnp.bfloat16).reshape(_N, _HW)
        )
        mask = (g2_ref[...] == c).astype(jnp.bfloat16)
        out = jax.lax.dot_general(
            sa_ref[...], mask,
            dimension_numbers=(((1,), (1,)), ((), ())),
            preferred_element_type=jnp.float32,
        )
        inter = out[0:_N, 0:_M]
        s_sum = out[0:_N, _M:_M + 1]
        g_sum = out[_N:_N + 1, 0:_M]
        union = s_sum + g_sum - inter
        return (inter + 1.0) / (union + 1.0)

    ratio = channel(1, s1_ref, 0)
    ratio += channel(2, s2_ref, 0)
    ratio += channel(3, s2_ref, 1)
    ratio += channel(4, s4_ref, 0)
    ratio += channel(5, s4_ref, 1)
    ratio += channel(6, s4_ref, 2)
    ratio += channel(7, s4_ref, 3)

    cost = 1.0 - ratio * (1.0 / _NC)
    minv = jnp.min(cost, axis=0, keepdims=True)
    iota_n = jax.lax.broadcasted_iota(jnp.int32, (_N, _M), 0)
    cand = jnp.where(cost <= minv, iota_n, _N)
    idx = jnp.min(cand, axis=0, keepdims=True)
    onehot = (iota_n == idx).astype(jnp.float32)

    pg = pg_ref[pl.ds(b, 1), :]
    seg_b = jnp.sum(minv * pg)

    target = jnp.sum(onehot * pg, axis=1, keepdims=True)
    p = prob_ref[pl.ds(b, 1), :].reshape(_N, 1)
    safe_t = jnp.where(target > 0, target, 1.0)
    kl_elem = jnp.where(
        target > 0, target * (jnp.log(safe_t) - jnp.log(p + 1e-8)), 0.0
    )
    kl_b = jnp.sum(kl_elem)

    @pl.when(b == 0)
    def _():
        acc_ref[0] = seg_b
        acc_ref[1] = kl_b

    @pl.when(b != 0)
    def _():
        acc_ref[0] += seg_b
        acc_ref[1] += kl_b

    @pl.when(b == _B - 1)
    def _():
        seg_loss = acc_ref[0] * (1.0 / _B)
        kl_loss = acc_ref[1] * (1.0 / (_B * _N))
        out_ref[0] = seg_loss + kl_loss
        out_ref[1] = seg_loss
        out_ref[2] = kl_loss


def kernel(gt_arr, sample_arr, prob, prob_gt, sample_shape):
    del sample_shape
    out = pl.pallas_call(
        _ot_kernel,
        grid=(_B,),
        in_specs=[
            pl.BlockSpec((1, _M, _H, _W), lambda b: (b, 0, 0, 0)),
            pl.BlockSpec((1, _N, 1, _H, _W), lambda b: (b, 0, 1, 0, 0)),
            pl.BlockSpec((1, _N, 2, _H, _W), lambda b: (b, 0, 1, 0, 0)),
            pl.BlockSpec((1, _N, 4, _H, _W), lambda b: (b, 0, 1, 0, 0)),
            pl.BlockSpec((_B, _N), lambda b: (0, 0)),
            pl.BlockSpec((_B, _M), lambda b: (0, 0)),
        ],
        out_specs=pl.BlockSpec(memory_space=pltpu.SMEM),
        out_shape=jax.ShapeDtypeStruct((3,), jnp.float32),
        scratch_shapes=[
            pltpu.VMEM((_MA, _HW), jnp.int32),
            pltpu.VMEM((_NA, _HW), jnp.bfloat16),
            pltpu.SMEM((2,), jnp.float32),
        ],
    )(gt_arr, sample_arr, sample_arr, sample_arr, prob, prob_gt)
    return (out[0], out[1], out[2])
